# baseline (speedup 1.0000x reference)
import jax
import jax.numpy as jnp
from jax.experimental import pallas as pl
from jax.experimental.pallas import tpu as pltpu

_TILE = 256

_DN_T = (((1,), (1,)), ((), ()))


def _mlp_kernel(x_ref, w1_ref, b1_ref, w2_ref, b2_ref, w3_ref, b3_ref,
                o_ref, w1bf_ref, xb_ref, h1_ref):
    i = pl.program_id(0)
    n = pl.num_programs(0) - 2

    @pl.when(i == 0)
    def _():
        w1bf_ref[...] = w1_ref[...].astype(jnp.bfloat16)

    @pl.when(i < n)
    def _():
        xt = x_ref[...]
        xb_ref[i % 2] = xt.astype(jnp.bfloat16).reshape(xt.shape[0], -1)

    @pl.when(jnp.logical_and(i >= 1, i < n + 1))
    def _():
        h1_ref[(i - 1) % 2] = jax.lax.dot_general(
            xb_ref[(i - 1) % 2], w1bf_ref[...], _DN_T,
            preferred_element_type=jnp.float32)

    @pl.when(i >= 2)
    def _():
        h1 = jnp.maximum(h1_ref[i % 2] + b1_ref[...], 0.0).astype(jnp.bfloat16)
        h2 = jnp.dot(h1, w2_ref[...].astype(jnp.bfloat16),
                     preferred_element_type=jnp.float32)
        h2 = jnp.maximum(h2 + b2_ref[...], 0.0).astype(jnp.bfloat16)
        out = jnp.dot(h2, w3_ref[...].astype(jnp.bfloat16),
                      preferred_element_type=jnp.float32)
        o_ref[...] = out + b3_ref[...]


def kernel(field, W1, b1, W2, b2, W3, b3):
    B = field.shape[0]
    nf, nl = field.shape[1], field.shape[2]
    in_dim = nf * nl
    hid1 = W1.shape[0]
    hid2 = W2.shape[0]

    w2t = W2.T
    w3t = W3.T
    b1r = b1.reshape(1, hid1)
    b2r = b2.reshape(1, hid2)
    b3r = b3.reshape(1, 1)

    n = B // _TILE
    grid = (n + 2,)
    out = pl.pallas_call(
        _mlp_kernel,
        grid=grid,
        in_specs=[
            pl.BlockSpec((_TILE, nf, nl), lambda i, n=n: (min(i, n - 1) if isinstance(i, int) else jnp.minimum(i, n - 1), 0, 0)),
            pl.BlockSpec((hid1, in_dim), lambda i: (0, 0)),
            pl.BlockSpec((1, hid1), lambda i: (0, 0)),
            pl.BlockSpec((hid1, hid2), lambda i: (0, 0)),
            pl.BlockSpec((1, hid2), lambda i: (0, 0)),
            pl.BlockSpec((hid2, 1), lambda i: (0, 0)),
            pl.BlockSpec((1, 1), lambda i: (0, 0)),
        ],
        out_specs=pl.BlockSpec(
            (_TILE, 1),
            lambda i: (max(i - 2, 0) if isinstance(i, int) else jnp.maximum(i - 2, 0), 0)),
        out_shape=jax.ShapeDtypeStruct((B, 1), jnp.float32),
        scratch_shapes=[
            pltpu.VMEM((hid1, in_dim), jnp.bfloat16),
            pltpu.VMEM((2, _TILE, in_dim), jnp.bfloat16),
            pltpu.VMEM((2, _TILE, hid1), jnp.float32),
        ],
    )(field, W1, b1r, w2t, b2r, w3t, b3r)
    return out
